# baseline (speedup 1.0000x reference)
import functools

import jax
import jax.numpy as jnp
from jax import lax
from jax.experimental import pallas as pl
from jax.experimental.pallas import tpu as pltpu
from jax.experimental.pallas import tpu_sc as plsc

N0, N1, N2, NNET = 10000, 2500, 625, 8000
D = 128
NW = 32
CHUNK = 128
EALIGN = NW * CHUNK


def _ceil_to(x, m):
    return (x + m - 1) // m * m


@functools.partial(jax.jit, static_argnums=(3,))
def _sc_segsum(feat, src1d, dst1d, nd_acc):
    c_tot = src1d.shape[0] // CHUNK
    cpw = c_tot // NW
    zc = nd_acc // 16 // 128
    mesh = plsc.VectorSubcoreMesh(core_axis_name="c", subcore_axis_name="s")

    @functools.partial(
        pl.kernel,
        out_type=jax.ShapeDtypeStruct((2, nd_acc, 128), jnp.float32),
        mesh=mesh,
        compiler_params=pltpu.CompilerParams(needs_layout_passes=False),
        scratch_types=[
            pltpu.VMEM((CHUNK,), jnp.int32),
            pltpu.VMEM((CHUNK,), jnp.int32),
            pltpu.VMEM((CHUNK, 128), jnp.float32),
            pltpu.VMEM((128, 128), jnp.float32),
            pltpu.VMEM_SHARED((nd_acc, 128), jnp.float32),
            pltpu.SemaphoreType.DMA,
        ],
    )
    def k(feat_h, src_h, dst_h, out_h,
          src_v, dst_v, rows_v, zero_v, acc_s, sem):
        c = lax.axis_index("c")
        s = lax.axis_index("s")
        w = c * 16 + s
        z16 = jnp.zeros((16,), jnp.float32)

        def zrow(i, carry):
            for j in range(8):
                zero_v[i, pl.ds(j * 16, 16)] = z16
            return carry
        lax.fori_loop(0, 128, zrow, 0)

        zb = s * (nd_acc // 16)
        for q in range(zc):
            pltpu.sync_copy(zero_v, acc_s.at[pl.ds(zb + q * 128, 128)])
        plsc.subcore_barrier()

        def body(j, carry):
            base = (w * cpw + j) * CHUNK
            pltpu.sync_copy(src_h.at[pl.ds(base, CHUNK)], src_v)
            pltpu.sync_copy(dst_h.at[pl.ds(base, CHUNK)], dst_v)
            pltpu.async_copy(feat_h.at[src_v], rows_v, sem).wait()
            pltpu.sync_copy(rows_v, acc_s.at[dst_v], add=True)
            return carry
        lax.fori_loop(0, cpw, body, 0)
        plsc.subcore_barrier()

        orows = nd_acc // 16
        pltpu.sync_copy(acc_s.at[pl.ds(s * orows, orows)],
                        out_h.at[c, pl.ds(s * orows, orows)])

    return k(feat, src1d, dst1d)


NDEG = 36864
NDEGR = NDEG // 128


@jax.jit
def _sc_degrees(dst1d):
    c_tot = dst1d.shape[0] // CHUNK
    cpw = c_tot // NW
    mesh = plsc.VectorSubcoreMesh(core_axis_name="c", subcore_axis_name="s")

    @functools.partial(
        pl.kernel,
        out_type=jax.ShapeDtypeStruct((2, 16, NDEGR, 128), jnp.float32),
        mesh=mesh,
        compiler_params=pltpu.CompilerParams(needs_layout_passes=False),
        scratch_types=[
            pltpu.VMEM((CHUNK,), jnp.int32),
            pltpu.VMEM((NDEGR, 128), jnp.float32),
        ],
    )
    def k(dst_h, deg_h, dst_v, degl_v):
        c = lax.axis_index("c")
        s = lax.axis_index("s")
        w = c * 16 + s
        z16 = jnp.zeros((16,), jnp.float32)
        one16 = jnp.ones((16,), jnp.float32)

        def zrow(i, carry):
            for j in range(8):
                degl_v[i, pl.ds(j * 16, 16)] = z16
            return carry
        lax.fori_loop(0, NDEGR, zrow, 0)

        def body(j, carry):
            base = (w * cpw + j) * CHUNK
            pltpu.sync_copy(dst_h.at[pl.ds(base, CHUNK)], dst_v)
            for l in range(CHUNK // 16):
                dv = dst_v[pl.ds(l * 16, 16)]
                plsc.addupdate_scatter(
                    degl_v,
                    [lax.shift_right_logical(dv, 7), lax.bitwise_and(dv, 127)],
                    one16)
            return carry
        lax.fori_loop(0, cpw, body, 0)
        pltpu.sync_copy(degl_v, deg_h.at[c, s])

    return k(dst1d)


SEG = 1024
NNET_PAD = 8192
NPT = NNET_PAD // NW


@jax.jit
def _sc_segmax(feat, src1d, dst1d):
    nseg = src1d.shape[0] // SEG
    mesh = plsc.VectorSubcoreMesh(core_axis_name="c", subcore_axis_name="s")

    @functools.partial(
        pl.kernel,
        out_type=jax.ShapeDtypeStruct((NNET_PAD, 128), jnp.float32),
        mesh=mesh,
        compiler_params=pltpu.CompilerParams(needs_layout_passes=False),
        scratch_types=[
            pltpu.VMEM((SEG,), jnp.int32),
            pltpu.VMEM((SEG,), jnp.int32),
            pltpu.VMEM((SEG + 16,), jnp.int32),
            pltpu.VMEM((SEG + 16,), jnp.int32),
            pltpu.VMEM((128, 128), jnp.float32),
            pltpu.VMEM((NPT, 128), jnp.float32),
            pltpu.SemaphoreType.DMA,
        ],
    )
    def k(feat_h, src_h, dst_h, out_h,
          dseg_v, sseg_v, cd_v, cs_v, rows_v, acc_v, sem):
        c = lax.axis_index("c")
        s = lax.axis_index("s")
        w = c * 16 + s
        lo = w * NPT
        neg = jnp.full((16,), -jnp.inf, dtype=jnp.float32)
        z16i = jnp.zeros((16,), jnp.int32)
        i16 = lax.iota(jnp.int32, 16)

        def initr(i, carry):
            for j in range(8):
                acc_v[i, pl.ds(j * 16, 16)] = neg
            return carry
        lax.fori_loop(0, NPT, initr, 0)

        def initc(i, carry):
            cd_v[pl.ds(i * 16, 16)] = z16i
            cs_v[pl.ds(i * 16, 16)] = i16 + i * 16
            return carry
        lax.fori_loop(0, (SEG + 16) // 16, initc, 0)

        def seg_body(g, carry):
            pltpu.sync_copy(dst_h.at[pl.ds(g * SEG, SEG)], dseg_v)
            pltpu.sync_copy(src_h.at[pl.ds(g * SEG, SEG)], sseg_v)

            def comp(t, n):
                dv = dseg_v[pl.ds(t * 16, 16)]
                sv = sseg_v[pl.ds(t * 16, 16)]
                m = jnp.logical_and(dv >= lo, dv < lo + NPT)
                cum = plsc.cumsum(m.astype(jnp.int32))
                pos = cum + (n - 1)
                plsc.store_scatter(cd_v, [pos], dv, mask=m)
                plsc.store_scatter(cs_v, [pos], sv, mask=m)
                return n + jnp.max(cum)
            n = lax.fori_loop(0, SEG // 16, comp, jnp.int32(0))

            def blk(b, carry2):
                pltpu.async_copy(feat_h.at[cs_v.at[pl.ds(b * 128, 128)]],
                                 rows_v, sem).wait()

                def edge(e2, carry3):
                    e = b * 128 + e2

                    @pl.when(e < n)
                    def _():
                        gv = cd_v[pl.ds(lax.shift_right_logical(e, 4) * 16, 16)]
                        lane = lax.bitwise_and(e, 15)
                        d = jnp.max(jnp.where(i16 == lane, gv,
                                              jnp.int32(-2147483648)))
                        r = d - lo
                        for u in range(8):
                            acc_v[r, pl.ds(u * 16, 16)] = jnp.maximum(
                                acc_v[r, pl.ds(u * 16, 16)],
                                rows_v[e2, pl.ds(u * 16, 16)])
                    return carry3
                lax.fori_loop(0, 128, edge, 0)
                return carry2
            lax.fori_loop(0, (n + 127) // 128, blk, 0)
            return carry
        lax.fori_loop(0, nseg, seg_body, 0)

        big_neg = jnp.float32(-3e38)

        def finr(i, carry):
            for j in range(8):
                v = acc_v[i, pl.ds(j * 16, 16)]
                acc_v[i, pl.ds(j * 16, 16)] = jnp.where(v > big_neg, v, 0.0)
            return carry
        lax.fori_loop(0, NPT, finr, 0)
        pltpu.sync_copy(acc_v, out_h.at[pl.ds(lo, NPT)])

    return k(feat, src1d, dst1d)


BN = 1024


def _mm_body(x_ref, w_ref, o_ref):
    o_ref[...] = jnp.dot(x_ref[...], w_ref[...],
                         preferred_element_type=jnp.float32)


def _tc_matmul(x, w):
    n, kd = x.shape
    m = w.shape[1]
    return pl.pallas_call(
        _mm_body,
        grid=(n // BN,),
        in_specs=[pl.BlockSpec((BN, kd), lambda i: (i, 0)),
                  pl.BlockSpec((kd, m), lambda i: (0, 0))],
        out_specs=pl.BlockSpec((BN, m), lambda i: (i, 0)),
        out_shape=jax.ShapeDtypeStruct((n, m), jnp.float32),
    )(x, w)


def _combine_body(sx_ref, p_ref, dp_ref, b_ref, o_ref):
    ps = p_ref[0] + p_ref[1]
    dg = jnp.maximum(jnp.sum(dp_ref[...], axis=(0, 1)), 1.0)
    ps3 = ps.reshape(BN // 128, 128, 128) / dg[:, :, None]
    o_ref[...] = jnp.tanh(sx_ref[...] + ps3.reshape(BN, 128) + b_ref[...])


def _tc_combine(sx, p, dp, base, b):
    n = sx.shape[0]
    bb = base // BN
    return pl.pallas_call(
        _combine_body,
        grid=(n // BN,),
        in_specs=[
            pl.BlockSpec((BN, 128), lambda i: (i, 0)),
            pl.BlockSpec((2, BN, 128), lambda i: (0, i, 0)),
            pl.BlockSpec((2, 16, BN // 128, 128), lambda i, b=bb: (0, 0, b + i, 0)),
            pl.BlockSpec((1, 128), lambda i: (0, 0)),
        ],
        out_specs=pl.BlockSpec((BN, 128), lambda i: (i, 0)),
        out_shape=jax.ShapeDtypeStruct((n, 128), jnp.float32),
    )(sx, p, dp, b.reshape(1, 128))


def _meanmm_body(p_ref, dp_ref, w_ref, o1_ref, o2_ref):
    ps = p_ref[0] + p_ref[1]
    dg = jnp.maximum(jnp.sum(dp_ref[...], axis=(0, 1)), 1.0)
    mfeat = (ps.reshape(BN // 128, 128, 128) / dg[:, :, None]).reshape(BN, 128)
    o1_ref[...] = mfeat
    o2_ref[...] = jnp.dot(mfeat, w_ref[...], preferred_element_type=jnp.float32)


def _tc_meanmm(p, dp, base, w, n):
    m = w.shape[1]
    bb = base // BN
    return pl.pallas_call(
        _meanmm_body,
        grid=(n // BN,),
        in_specs=[
            pl.BlockSpec((2, BN, 128), lambda i: (0, i, 0)),
            pl.BlockSpec((2, 16, BN // 128, 128), lambda i, b=bb: (0, 0, b + i, 0)),
            pl.BlockSpec((128, m), lambda i: (0, 0)),
        ],
        out_specs=[pl.BlockSpec((BN, 128), lambda i: (i, 0)),
                   pl.BlockSpec((BN, m), lambda i: (i, 0))],
        out_shape=[jax.ShapeDtypeStruct((n, 128), jnp.float32),
                   jax.ShapeDtypeStruct((n, m), jnp.float32)],
    )(p, dp, w)


def _upmm_body(a_ref, x_ref, cp_ref, wt_ref, wb_ref, o_ref):
    asum = a_ref[0] + a_ref[1]
    cnt = jnp.sum(cp_ref[...], axis=(0, 1))
    xc = (x_ref[...].reshape(BN // 128, 128, 128) * cnt[:, :, None]
          ).reshape(BN, 128)
    o_ref[...] = (jnp.dot(asum, wt_ref[...], preferred_element_type=jnp.float32)
                  + jnp.dot(xc, wb_ref[...], preferred_element_type=jnp.float32))


def _tc_upmm(p, x, cp, base, wt, wb):
    n = x.shape[0]
    m = wt.shape[1]
    bb = base // BN
    return pl.pallas_call(
        _upmm_body,
        grid=(n // BN,),
        in_specs=[
            pl.BlockSpec((2, BN, 128), lambda i: (0, i, 0)),
            pl.BlockSpec((BN, 128), lambda i: (i, 0)),
            pl.BlockSpec((2, 16, BN // 128, 128), lambda i, b=bb: (0, 0, b + i, 0)),
            pl.BlockSpec((128, m), lambda i: (0, 0)),
            pl.BlockSpec((128, m), lambda i: (0, 0)),
        ],
        out_specs=pl.BlockSpec((BN, m), lambda i: (i, 0)),
        out_shape=jax.ShapeDtypeStruct((n, m), jnp.float32),
    )(p, x, cp, wt, wb)


def _mlp_body(y_ref, w0_ref, b0_ref, w1_ref, b1_ref, o_ref):
    h = jnp.tanh(jnp.dot(y_ref[...], w0_ref[...],
                         preferred_element_type=jnp.float32) + b0_ref[...])
    o_ref[...] = jnp.dot(h, w1_ref[...],
                         preferred_element_type=jnp.float32) + b1_ref[...]


def _tc_mlp(y, w0, b0, w1, b1):
    n = y.shape[0]
    return pl.pallas_call(
        _mlp_body,
        grid=(n // BN,),
        in_specs=[
            pl.BlockSpec((BN, 128), lambda i: (i, 0)),
            pl.BlockSpec((128, 128), lambda i: (0, 0)),
            pl.BlockSpec((1, 128), lambda i: (0, 0)),
            pl.BlockSpec((128, 1), lambda i: (0, 0)),
            pl.BlockSpec((1, 1), lambda i: (0, 0)),
        ],
        out_specs=pl.BlockSpec((BN, 1), lambda i: (i, 0)),
        out_shape=jax.ShapeDtypeStruct((n, 1), jnp.float32),
    )(y, w0, b0.reshape(1, 128), w1, b1.reshape(1, 1))


def _prep_edges(src, dst, nd):
    e = src.shape[0]
    ep = _ceil_to(e, EALIGN)
    pad = ep - e
    src_p = jnp.concatenate([src, jnp.zeros((pad,), jnp.int32)])
    dst_p = jnp.concatenate(
        [dst, nd + (jnp.arange(pad, dtype=jnp.int32) % 32)])
    return src_p, dst_p


def _prep_conn(src, dst):
    e = src.shape[0]
    ep = _ceil_to(e, SEG)
    pad = ep - e
    src_p = jnp.concatenate([src, jnp.zeros((pad,), jnp.int32)])
    dst_p = jnp.concatenate(
        [dst, jnp.full((pad,), 2 ** 30, dtype=jnp.int32)])
    return src_p, dst_p


def _pad_rows(a, n):
    return jnp.pad(a, ((0, n - a.shape[0]), (0, 0)))


def kernel(x, edge_lv0, d01_src, d01_dst, edge_lv1, d12_src, d12_dst,
           edge_lv2, u21_src, u21_dst, u10_src, u10_dst, conn_src, conn_dst,
           Ws0, Wn0, b0, Ws1, Wn1, b1, Ws2, Wn2, b2, Ws3, Wn3, b3,
           Ws4, Wn4, b4, Wm0, bm0, Wm1, bm1):
    NA0, NA1, NA2 = 10240, 4096, 2048
    NP1, NP2 = 3072, 1024
    NP0 = 10240

    e0 = _prep_edges(edge_lv0[0], edge_lv0[1], N0)
    e1 = _prep_edges(edge_lv1[0], edge_lv1[1], N1)
    e2 = _prep_edges(edge_lv2[0], edge_lv2[1], N2)
    ed01 = _prep_edges(d01_src, d01_dst, N1)
    ed12 = _prep_edges(d12_src, d12_dst, N2)
    eu21 = _prep_edges(u21_src, u21_dst, N1)
    eu10 = _prep_edges(u10_src, u10_dst, N0)
    cn = _prep_conn(conn_src, conn_dst)

    alldst = jnp.concatenate(
        [e0[1], ed01[1] + 10240, e1[1] + 14336, ed12[1] + 18432,
         e2[1] + 20480, eu21[1] + 22528, eu10[1] + 26624])
    DEG = _sc_degrees(alldst)
    B_LV0, B_D01, B_LV1, B_D12 = 0, 10240, 14336, 18432
    B_LV2, B_U21, B_U10 = 20480, 22528, 26624

    xp = _pad_rows(x, NP0)
    t0 = _tc_matmul(xp, jnp.concatenate([Wn0, Ws0], axis=1))
    xw0, xs0 = t0[:, :128], t0[:, 128:]
    Pa = _sc_segsum(xw0, e0[0], e0[1], NA0)
    x0 = _tc_combine(xs0, Pa, DEG, B_LV0, b0)

    Pb = _sc_segsum(x0, ed01[0], ed01[1], NA1)
    p1, pw1 = _tc_meanmm(Pb, DEG, B_D01, jnp.concatenate([Wn1, Ws1], axis=1), NP1)
    Pc = _sc_segsum(pw1[:, :128], e1[0], e1[1], NA1)
    x1 = _tc_combine(pw1[:, 128:], Pc, DEG, B_LV1, b1)

    Pd = _sc_segsum(x1, ed12[0], ed12[1], NA2)
    p2, pw2 = _tc_meanmm(Pd, DEG, B_D12, jnp.concatenate([Wn2, Ws2], axis=1), NP2)
    Pe = _sc_segsum(pw2[:, :128], e2[0], e2[1], NA2)
    x2 = _tc_combine(pw2[:, 128:], Pe, DEG, B_LV2, b2)

    Pf = _sc_segsum(x2, eu21[0], eu21[1], NA1)
    wt3 = jnp.concatenate([Wn3[:128], Ws3[:128]], axis=1)
    wb3 = jnp.concatenate([Wn3[128:], Ws3[128:]], axis=1)
    hh1 = _tc_upmm(Pf, x1, DEG, B_U21, wt3, wb3)
    Pg = _sc_segsum(hh1[:, :128], e1[0], e1[1], NA1)
    x1u = _tc_combine(hh1[:, 128:], Pg, DEG, B_LV1, b3)

    Ph = _sc_segsum(x1u, eu10[0], eu10[1], NA0)
    wt4 = jnp.concatenate([Wn4[:128], Ws4[:128]], axis=1)
    wb4 = jnp.concatenate([Wn4[128:], Ws4[128:]], axis=1)
    hh0 = _tc_upmm(Ph, x0, DEG, B_U10, wt4, wb4)
    Pi = _sc_segsum(hh0[:, :128], e0[0], e0[1], NA0)
    x0u = _tc_combine(hh0[:, 128:], Pi, DEG, B_LV0, b4)

    y = _sc_segmax(x0u, cn[0], cn[1])
    out = _tc_mlp(y, Wm0, bm0, Wm1, bm1)
    return out[:NNET]


# speedup vs baseline: 1.1418x; 1.1418x over previous
import functools

import jax
import jax.numpy as jnp
from jax import lax
from jax.experimental import pallas as pl
from jax.experimental.pallas import tpu as pltpu
from jax.experimental.pallas import tpu_sc as plsc

N0, N1, N2, NNET = 10000, 2500, 625, 8000
D = 128
NW = 32
CHUNK = 128
EALIGN = NW * CHUNK


def _ceil_to(x, m):
    return (x + m - 1) // m * m


@functools.partial(jax.jit, static_argnums=(3,))
def _sc_segsum(feat, src1d, dst1d, nd_acc):
    c_tot = src1d.shape[0] // CHUNK
    cpw = c_tot // NW
    zc = nd_acc // 16 // 64
    mesh = plsc.VectorSubcoreMesh(core_axis_name="c", subcore_axis_name="s")

    @functools.partial(
        pl.kernel,
        out_type=jax.ShapeDtypeStruct((2, nd_acc, 128), jnp.float32),
        mesh=mesh,
        compiler_params=pltpu.CompilerParams(needs_layout_passes=False),
        scratch_types=[
            pltpu.VMEM((CHUNK,), jnp.int32),
            pltpu.VMEM((CHUNK,), jnp.int32),
            pltpu.VMEM((CHUNK,), jnp.int32),
            pltpu.VMEM((CHUNK,), jnp.int32),
            pltpu.VMEM((CHUNK, 128), jnp.float32),
            pltpu.VMEM((CHUNK, 128), jnp.float32),
            pltpu.VMEM((64, 128), jnp.float32),
            pltpu.VMEM_SHARED((nd_acc, 128), jnp.float32),
            pltpu.SemaphoreType.DMA,
            pltpu.SemaphoreType.DMA,
        ],
    )
    def k(feat_h, src_h, dst_h, out_h,
          src0_v, src1_v, dst0_v, dst1_v, rows0_v, rows1_v, zero_v, acc_s,
          sem0, sem1):
        c = lax.axis_index("c")
        s = lax.axis_index("s")
        w = c * 16 + s
        z16 = jnp.zeros((16,), jnp.float32)
        ebase = w * (cpw * CHUNK)

        def zrow(i, carry):
            for j in range(8):
                zero_v[i, pl.ds(j * 16, 16)] = z16
            return carry
        lax.fori_loop(0, 64, zrow, 0)

        zb = s * (nd_acc // 16)
        for q in range(zc):
            pltpu.sync_copy(zero_v, acc_s.at[pl.ds(zb + q * 64, 64)])
        plsc.subcore_barrier()

        bufs = (rows0_v, rows1_v)
        srcs = (src0_v, src1_v)
        dsts = (dst0_v, dst1_v)
        sems = (sem0, sem1)
        pltpu.sync_copy(src_h.at[pl.ds(ebase, CHUNK)], src0_v)
        pend = {0: pltpu.async_copy(feat_h.at[src0_v], rows0_v, sem0)}
        for j in range(cpw):
            b = j % 2
            if j + 1 < cpw:
                nb = (j + 1) % 2
                pltpu.sync_copy(
                    dst_h.at[pl.ds(ebase + (j + 1) * CHUNK, CHUNK)] if False
                    else src_h.at[pl.ds(ebase + (j + 1) * CHUNK, CHUNK)],
                    srcs[nb])
                pend[j + 1] = pltpu.async_copy(
                    feat_h.at[srcs[nb]], bufs[nb], sems[nb])
            pltpu.sync_copy(dst_h.at[pl.ds(ebase + j * CHUNK, CHUNK)],
                            dsts[b])
            pend[j].wait()
            pltpu.sync_copy(bufs[b], acc_s.at[dsts[b]], add=True)
        plsc.subcore_barrier()

        orows = nd_acc // 16
        pltpu.sync_copy(acc_s.at[pl.ds(s * orows, orows)],
                        out_h.at[c, pl.ds(s * orows, orows)])

    return k(feat, src1d, dst1d)


NDEG = 36864
NDEGR = NDEG // 128


@jax.jit
def _sc_degrees(dst1d):
    c_tot = dst1d.shape[0] // CHUNK
    cpw = c_tot // NW
    mesh = plsc.VectorSubcoreMesh(core_axis_name="c", subcore_axis_name="s")

    @functools.partial(
        pl.kernel,
        out_type=jax.ShapeDtypeStruct((2, 16, NDEGR, 128), jnp.float32),
        mesh=mesh,
        compiler_params=pltpu.CompilerParams(needs_layout_passes=False),
        scratch_types=[
            pltpu.VMEM((CHUNK,), jnp.int32),
            pltpu.VMEM((NDEGR, 128), jnp.float32),
        ],
    )
    def k(dst_h, deg_h, dst_v, degl_v):
        c = lax.axis_index("c")
        s = lax.axis_index("s")
        w = c * 16 + s
        z16 = jnp.zeros((16,), jnp.float32)
        one16 = jnp.ones((16,), jnp.float32)

        def zrow(i, carry):
            for j in range(8):
                degl_v[i, pl.ds(j * 16, 16)] = z16
            return carry
        lax.fori_loop(0, NDEGR, zrow, 0)

        def body(j, carry):
            base = (w * cpw + j) * CHUNK
            pltpu.sync_copy(dst_h.at[pl.ds(base, CHUNK)], dst_v)
            for l in range(CHUNK // 16):
                dv = dst_v[pl.ds(l * 16, 16)]
                plsc.addupdate_scatter(
                    degl_v,
                    [lax.shift_right_logical(dv, 7), lax.bitwise_and(dv, 127)],
                    one16)
            return carry
        lax.fori_loop(0, cpw, body, 0)
        pltpu.sync_copy(degl_v, deg_h.at[c, s])

    return k(dst1d)


SEG = 1024
NNET_PAD = 8192
NPT = NNET_PAD // NW


@jax.jit
def _sc_segmax(feat, src1d, dst1d):
    nseg = src1d.shape[0] // SEG
    mesh = plsc.VectorSubcoreMesh(core_axis_name="c", subcore_axis_name="s")

    @functools.partial(
        pl.kernel,
        out_type=jax.ShapeDtypeStruct((NNET_PAD, 128), jnp.float32),
        mesh=mesh,
        compiler_params=pltpu.CompilerParams(needs_layout_passes=False),
        scratch_types=[
            pltpu.VMEM((SEG,), jnp.int32),
            pltpu.VMEM((SEG,), jnp.int32),
            pltpu.VMEM((SEG + 16,), jnp.int32),
            pltpu.VMEM((SEG + 16,), jnp.int32),
            pltpu.VMEM((128, 128), jnp.float32),
            pltpu.VMEM((NPT, 128), jnp.float32),
            pltpu.SemaphoreType.DMA,
        ],
    )
    def k(feat_h, src_h, dst_h, out_h,
          dseg_v, sseg_v, cd_v, cs_v, rows_v, acc_v, sem):
        c = lax.axis_index("c")
        s = lax.axis_index("s")
        w = c * 16 + s
        lo = w * NPT
        neg = jnp.full((16,), -jnp.inf, dtype=jnp.float32)
        z16i = jnp.zeros((16,), jnp.int32)
        i16 = lax.iota(jnp.int32, 16)

        def initr(i, carry):
            for j in range(8):
                acc_v[i, pl.ds(j * 16, 16)] = neg
            return carry
        lax.fori_loop(0, NPT, initr, 0)

        def initc(i, carry):
            cd_v[pl.ds(i * 16, 16)] = z16i
            cs_v[pl.ds(i * 16, 16)] = i16 + i * 16
            return carry
        lax.fori_loop(0, (SEG + 16) // 16, initc, 0)

        def seg_body(g, carry):
            pltpu.sync_copy(dst_h.at[pl.ds(g * SEG, SEG)], dseg_v)
            pltpu.sync_copy(src_h.at[pl.ds(g * SEG, SEG)], sseg_v)

            def comp(t, n):
                dv = dseg_v[pl.ds(t * 16, 16)]
                sv = sseg_v[pl.ds(t * 16, 16)]
                m = jnp.logical_and(dv >= lo, dv < lo + NPT)
                cum = plsc.cumsum(m.astype(jnp.int32))
                pos = cum + (n - 1)
                plsc.store_scatter(cd_v, [pos], dv, mask=m)
                plsc.store_scatter(cs_v, [pos], sv, mask=m)
                return n + jnp.max(cum)
            n = lax.fori_loop(0, SEG // 16, comp, jnp.int32(0))

            def blk(b, carry2):
                pltpu.async_copy(feat_h.at[cs_v.at[pl.ds(b * 128, 128)]],
                                 rows_v, sem).wait()

                def edge(e2, carry3):
                    e = b * 128 + e2

                    @pl.when(e < n)
                    def _():
                        gv = cd_v[pl.ds(lax.shift_right_logical(e, 4) * 16, 16)]
                        lane = lax.bitwise_and(e, 15)
                        d = jnp.max(jnp.where(i16 == lane, gv,
                                              jnp.int32(-2147483648)))
                        r = d - lo
                        for u in range(8):
                            acc_v[r, pl.ds(u * 16, 16)] = jnp.maximum(
                                acc_v[r, pl.ds(u * 16, 16)],
                                rows_v[e2, pl.ds(u * 16, 16)])
                    return carry3
                lax.fori_loop(0, 128, edge, 0)
                return carry2
            lax.fori_loop(0, (n + 127) // 128, blk, 0)
            return carry
        lax.fori_loop(0, nseg, seg_body, 0)

        big_neg = jnp.float32(-3e38)

        def finr(i, carry):
            for j in range(8):
                v = acc_v[i, pl.ds(j * 16, 16)]
                acc_v[i, pl.ds(j * 16, 16)] = jnp.where(v > big_neg, v, 0.0)
            return carry
        lax.fori_loop(0, NPT, finr, 0)
        pltpu.sync_copy(acc_v, out_h.at[pl.ds(lo, NPT)])

    return k(feat, src1d, dst1d)


BN = 1024


def _mm_body(x_ref, w_ref, o_ref):
    o_ref[...] = jnp.dot(x_ref[...], w_ref[...],
                         preferred_element_type=jnp.float32)


def _tc_matmul(x, w):
    n, kd = x.shape
    m = w.shape[1]
    return pl.pallas_call(
        _mm_body,
        grid=(n // BN,),
        in_specs=[pl.BlockSpec((BN, kd), lambda i: (i, 0)),
                  pl.BlockSpec((kd, m), lambda i: (0, 0))],
        out_specs=pl.BlockSpec((BN, m), lambda i: (i, 0)),
        out_shape=jax.ShapeDtypeStruct((n, m), jnp.float32),
    )(x, w)


def _combine_body(sx_ref, p_ref, dp_ref, b_ref, o_ref):
    ps = p_ref[0] + p_ref[1]
    dg = jnp.maximum(jnp.sum(dp_ref[...], axis=(0, 1)), 1.0)
    ps3 = ps.reshape(BN // 128, 128, 128) / dg[:, :, None]
    o_ref[...] = jnp.tanh(sx_ref[...] + ps3.reshape(BN, 128) + b_ref[...])


def _tc_combine(sx, p, dp, base, b):
    n = sx.shape[0]
    bb = base // BN
    return pl.pallas_call(
        _combine_body,
        grid=(n // BN,),
        in_specs=[
            pl.BlockSpec((BN, 128), lambda i: (i, 0)),
            pl.BlockSpec((2, BN, 128), lambda i: (0, i, 0)),
            pl.BlockSpec((2, 16, BN // 128, 128), lambda i, b=bb: (0, 0, b + i, 0)),
            pl.BlockSpec((1, 128), lambda i: (0, 0)),
        ],
        out_specs=pl.BlockSpec((BN, 128), lambda i: (i, 0)),
        out_shape=jax.ShapeDtypeStruct((n, 128), jnp.float32),
    )(sx, p, dp, b.reshape(1, 128))


def _meanmm_body(p_ref, dp_ref, w_ref, o1_ref, o2_ref):
    ps = p_ref[0] + p_ref[1]
    dg = jnp.maximum(jnp.sum(dp_ref[...], axis=(0, 1)), 1.0)
    mfeat = (ps.reshape(BN // 128, 128, 128) / dg[:, :, None]).reshape(BN, 128)
    o1_ref[...] = mfeat
    o2_ref[...] = jnp.dot(mfeat, w_ref[...], preferred_element_type=jnp.float32)


def _tc_meanmm(p, dp, base, w, n):
    m = w.shape[1]
    bb = base // BN
    return pl.pallas_call(
        _meanmm_body,
        grid=(n // BN,),
        in_specs=[
            pl.BlockSpec((2, BN, 128), lambda i: (0, i, 0)),
            pl.BlockSpec((2, 16, BN // 128, 128), lambda i, b=bb: (0, 0, b + i, 0)),
            pl.BlockSpec((128, m), lambda i: (0, 0)),
        ],
        out_specs=[pl.BlockSpec((BN, 128), lambda i: (i, 0)),
                   pl.BlockSpec((BN, m), lambda i: (i, 0))],
        out_shape=[jax.ShapeDtypeStruct((n, 128), jnp.float32),
                   jax.ShapeDtypeStruct((n, m), jnp.float32)],
    )(p, dp, w)


def _upmm_body(a_ref, x_ref, cp_ref, wt_ref, wb_ref, o_ref):
    asum = a_ref[0] + a_ref[1]
    cnt = jnp.sum(cp_ref[...], axis=(0, 1))
    xc = (x_ref[...].reshape(BN // 128, 128, 128) * cnt[:, :, None]
          ).reshape(BN, 128)
    o_ref[...] = (jnp.dot(asum, wt_ref[...], preferred_element_type=jnp.float32)
                  + jnp.dot(xc, wb_ref[...], preferred_element_type=jnp.float32))


def _tc_upmm(p, x, cp, base, wt, wb):
    n = x.shape[0]
    m = wt.shape[1]
    bb = base // BN
    return pl.pallas_call(
        _upmm_body,
        grid=(n // BN,),
        in_specs=[
            pl.BlockSpec((2, BN, 128), lambda i: (0, i, 0)),
            pl.BlockSpec((BN, 128), lambda i: (i, 0)),
            pl.BlockSpec((2, 16, BN // 128, 128), lambda i, b=bb: (0, 0, b + i, 0)),
            pl.BlockSpec((128, m), lambda i: (0, 0)),
            pl.BlockSpec((128, m), lambda i: (0, 0)),
        ],
        out_specs=pl.BlockSpec((BN, m), lambda i: (i, 0)),
        out_shape=jax.ShapeDtypeStruct((n, m), jnp.float32),
    )(p, x, cp, wt, wb)


def _mlp_body(y_ref, w0_ref, b0_ref, w1_ref, b1_ref, o_ref):
    h = jnp.tanh(jnp.dot(y_ref[...], w0_ref[...],
                         preferred_element_type=jnp.float32) + b0_ref[...])
    o_ref[...] = jnp.dot(h, w1_ref[...],
                         preferred_element_type=jnp.float32) + b1_ref[...]


def _tc_mlp(y, w0, b0, w1, b1):
    n = y.shape[0]
    return pl.pallas_call(
        _mlp_body,
        grid=(n // BN,),
        in_specs=[
            pl.BlockSpec((BN, 128), lambda i: (i, 0)),
            pl.BlockSpec((128, 128), lambda i: (0, 0)),
            pl.BlockSpec((1, 128), lambda i: (0, 0)),
            pl.BlockSpec((128, 1), lambda i: (0, 0)),
            pl.BlockSpec((1, 1), lambda i: (0, 0)),
        ],
        out_specs=pl.BlockSpec((BN, 1), lambda i: (i, 0)),
        out_shape=jax.ShapeDtypeStruct((n, 1), jnp.float32),
    )(y, w0, b0.reshape(1, 128), w1, b1.reshape(1, 1))


def _prep_edges(src, dst, nd):
    e = src.shape[0]
    ep = _ceil_to(e, EALIGN)
    pad = ep - e
    src_p = jnp.concatenate([src, jnp.zeros((pad,), jnp.int32)])
    dst_p = jnp.concatenate(
        [dst, nd + (jnp.arange(pad, dtype=jnp.int32) % 32)])
    return src_p, dst_p


def _prep_conn(src, dst):
    e = src.shape[0]
    ep = _ceil_to(e, SEG)
    pad = ep - e
    src_p = jnp.concatenate([src, jnp.zeros((pad,), jnp.int32)])
    dst_p = jnp.concatenate(
        [dst, jnp.full((pad,), 2 ** 30, dtype=jnp.int32)])
    return src_p, dst_p


def _pad_rows(a, n):
    return jnp.pad(a, ((0, n - a.shape[0]), (0, 0)))


def kernel(x, edge_lv0, d01_src, d01_dst, edge_lv1, d12_src, d12_dst,
           edge_lv2, u21_src, u21_dst, u10_src, u10_dst, conn_src, conn_dst,
           Ws0, Wn0, b0, Ws1, Wn1, b1, Ws2, Wn2, b2, Ws3, Wn3, b3,
           Ws4, Wn4, b4, Wm0, bm0, Wm1, bm1):
    NA0, NA1, NA2 = 10240, 4096, 2048
    NP1, NP2 = 3072, 1024
    NP0 = 10240

    e0 = _prep_edges(edge_lv0[0], edge_lv0[1], N0)
    e1 = _prep_edges(edge_lv1[0], edge_lv1[1], N1)
    e2 = _prep_edges(edge_lv2[0], edge_lv2[1], N2)
    ed01 = _prep_edges(d01_src, d01_dst, N1)
    ed12 = _prep_edges(d12_src, d12_dst, N2)
    eu21 = _prep_edges(u21_src, u21_dst, N1)
    eu10 = _prep_edges(u10_src, u10_dst, N0)
    cn = _prep_conn(conn_src, conn_dst)

    alldst = jnp.concatenate(
        [e0[1], ed01[1] + 10240, e1[1] + 14336, ed12[1] + 18432,
         e2[1] + 20480, eu21[1] + 22528, eu10[1] + 26624])
    DEG = _sc_degrees(alldst)
    B_LV0, B_D01, B_LV1, B_D12 = 0, 10240, 14336, 18432
    B_LV2, B_U21, B_U10 = 20480, 22528, 26624

    xp = _pad_rows(x, NP0)
    t0 = _tc_matmul(xp, jnp.concatenate([Wn0, Ws0], axis=1))
    xw0, xs0 = t0[:, :128], t0[:, 128:]
    Pa = _sc_segsum(xw0, e0[0], e0[1], NA0)
    x0 = _tc_combine(xs0, Pa, DEG, B_LV0, b0)

    Pb = _sc_segsum(x0, ed01[0], ed01[1], NA1)
    p1, pw1 = _tc_meanmm(Pb, DEG, B_D01, jnp.concatenate([Wn1, Ws1], axis=1), NP1)
    Pc = _sc_segsum(pw1[:, :128], e1[0], e1[1], NA1)
    x1 = _tc_combine(pw1[:, 128:], Pc, DEG, B_LV1, b1)

    Pd = _sc_segsum(x1, ed12[0], ed12[1], NA2)
    p2, pw2 = _tc_meanmm(Pd, DEG, B_D12, jnp.concatenate([Wn2, Ws2], axis=1), NP2)
    Pe = _sc_segsum(pw2[:, :128], e2[0], e2[1], NA2)
    x2 = _tc_combine(pw2[:, 128:], Pe, DEG, B_LV2, b2)

    Pf = _sc_segsum(x2, eu21[0], eu21[1], NA1)
    wt3 = jnp.concatenate([Wn3[:128], Ws3[:128]], axis=1)
    wb3 = jnp.concatenate([Wn3[128:], Ws3[128:]], axis=1)
    hh1 = _tc_upmm(Pf, x1, DEG, B_U21, wt3, wb3)
    Pg = _sc_segsum(hh1[:, :128], e1[0], e1[1], NA1)
    x1u = _tc_combine(hh1[:, 128:], Pg, DEG, B_LV1, b3)

    Ph = _sc_segsum(x1u, eu10[0], eu10[1], NA0)
    wt4 = jnp.concatenate([Wn4[:128], Ws4[:128]], axis=1)
    wb4 = jnp.concatenate([Wn4[128:], Ws4[128:]], axis=1)
    hh0 = _tc_upmm(Ph, x0, DEG, B_U10, wt4, wb4)
    Pi = _sc_segsum(hh0[:, :128], e0[0], e0[1], NA0)
    x0u = _tc_combine(hh0[:, 128:], Pi, DEG, B_LV0, b4)

    y = _sc_segmax(x0u, cn[0], cn[1])
    out = _tc_mlp(y, Wm0, bm0, Wm1, bm1)
    return out[:NNET]


# speedup vs baseline: 1.2945x; 1.1337x over previous
import functools

import jax
import jax.numpy as jnp
from jax import lax
from jax.experimental import pallas as pl
from jax.experimental.pallas import tpu as pltpu
from jax.experimental.pallas import tpu_sc as plsc

N0, N1, N2, NNET = 10000, 2500, 625, 8000
D = 128
NW = 32
CHUNK = 128
EALIGN = NW * CHUNK


def _ceil_to(x, m):
    return (x + m - 1) // m * m


@functools.partial(jax.jit, static_argnums=(3,))
def _sc_segsum(feat, src1d, dst1d, nd_acc):
    c_tot = src1d.shape[0] // CHUNK
    cpw = c_tot // NW
    zc = nd_acc // 16 // 64
    mesh = plsc.VectorSubcoreMesh(core_axis_name="c", subcore_axis_name="s")

    @functools.partial(
        pl.kernel,
        out_type=jax.ShapeDtypeStruct((2, nd_acc, 128), jnp.float32),
        mesh=mesh,
        compiler_params=pltpu.CompilerParams(needs_layout_passes=False),
        scratch_types=[
            pltpu.VMEM((CHUNK,), jnp.int32),
            pltpu.VMEM((CHUNK,), jnp.int32),
            pltpu.VMEM((CHUNK,), jnp.int32),
            pltpu.VMEM((CHUNK,), jnp.int32),
            pltpu.VMEM((CHUNK, 128), jnp.float32),
            pltpu.VMEM((CHUNK, 128), jnp.float32),
            pltpu.VMEM((64, 128), jnp.float32),
            pltpu.VMEM_SHARED((nd_acc, 128), jnp.float32),
            pltpu.SemaphoreType.DMA,
            pltpu.SemaphoreType.DMA,
        ],
    )
    def k(feat_h, src_h, dst_h, out_h,
          src0_v, src1_v, dst0_v, dst1_v, rows0_v, rows1_v, zero_v, acc_s,
          sem0, sem1):
        c = lax.axis_index("c")
        s = lax.axis_index("s")
        w = c * 16 + s
        z16 = jnp.zeros((16,), jnp.float32)
        ebase = w * (cpw * CHUNK)

        def zrow(i, carry):
            for j in range(8):
                zero_v[i, pl.ds(j * 16, 16)] = z16
            return carry
        lax.fori_loop(0, 64, zrow, 0)

        zb = s * (nd_acc // 16)
        for q in range(zc):
            pltpu.sync_copy(zero_v, acc_s.at[pl.ds(zb + q * 64, 64)])
        plsc.subcore_barrier()

        bufs = (rows0_v, rows1_v)
        srcs = (src0_v, src1_v)
        dsts = (dst0_v, dst1_v)
        sems = (sem0, sem1)
        pltpu.sync_copy(src_h.at[pl.ds(ebase, CHUNK)], src0_v)
        pend = {0: pltpu.async_copy(feat_h.at[src0_v], rows0_v, sem0)}
        for j in range(cpw):
            b = j % 2
            if j + 1 < cpw:
                nb = (j + 1) % 2
                pltpu.sync_copy(
                    dst_h.at[pl.ds(ebase + (j + 1) * CHUNK, CHUNK)] if False
                    else src_h.at[pl.ds(ebase + (j + 1) * CHUNK, CHUNK)],
                    srcs[nb])
                pend[j + 1] = pltpu.async_copy(
                    feat_h.at[srcs[nb]], bufs[nb], sems[nb])
            pltpu.sync_copy(dst_h.at[pl.ds(ebase + j * CHUNK, CHUNK)],
                            dsts[b])
            pend[j].wait()
            pltpu.sync_copy(bufs[b], acc_s.at[dsts[b]], add=True)
        plsc.subcore_barrier()

        orows = nd_acc // 16
        pltpu.sync_copy(acc_s.at[pl.ds(s * orows, orows)],
                        out_h.at[c, pl.ds(s * orows, orows)])

    return k(feat, src1d, dst1d)


NDEG = 36864
NDEGR = NDEG // 128


@jax.jit
def _sc_degrees(dst1d):
    c_tot = dst1d.shape[0] // CHUNK
    cpw = c_tot // NW
    mesh = plsc.VectorSubcoreMesh(core_axis_name="c", subcore_axis_name="s")

    @functools.partial(
        pl.kernel,
        out_type=jax.ShapeDtypeStruct((2, 16, NDEGR, 128), jnp.float32),
        mesh=mesh,
        compiler_params=pltpu.CompilerParams(needs_layout_passes=False),
        scratch_types=[
            pltpu.VMEM((CHUNK,), jnp.int32),
            pltpu.VMEM((NDEGR, 128), jnp.float32),
        ],
    )
    def k(dst_h, deg_h, dst_v, degl_v):
        c = lax.axis_index("c")
        s = lax.axis_index("s")
        w = c * 16 + s
        z16 = jnp.zeros((16,), jnp.float32)
        one16 = jnp.ones((16,), jnp.float32)

        def zrow(i, carry):
            for j in range(8):
                degl_v[i, pl.ds(j * 16, 16)] = z16
            return carry
        lax.fori_loop(0, NDEGR, zrow, 0)

        def body(j, carry):
            base = (w * cpw + j) * CHUNK
            pltpu.sync_copy(dst_h.at[pl.ds(base, CHUNK)], dst_v)
            for l in range(CHUNK // 16):
                dv = dst_v[pl.ds(l * 16, 16)]
                plsc.addupdate_scatter(
                    degl_v,
                    [lax.shift_right_logical(dv, 7), lax.bitwise_and(dv, 127)],
                    one16)
            return carry
        lax.fori_loop(0, cpw, body, 0)
        pltpu.sync_copy(degl_v, deg_h.at[c, s])

    return k(dst1d)


SEG = 1024
NNET_PAD = 8192
NPT = NNET_PAD // 16


@jax.jit
def _sc_segmax(feat, src1d, dst1d):
    nseg = src1d.shape[0] // SEG
    nsegh = nseg // 2
    mesh = plsc.VectorSubcoreMesh(core_axis_name="c", subcore_axis_name="s")

    @functools.partial(
        pl.kernel,
        out_type=jax.ShapeDtypeStruct((2, NNET_PAD, 128), jnp.float32),
        mesh=mesh,
        compiler_params=pltpu.CompilerParams(needs_layout_passes=False),
        scratch_types=[
            pltpu.VMEM((SEG,), jnp.int32),
            pltpu.VMEM((SEG,), jnp.int32),
            pltpu.VMEM((SEG + 16,), jnp.int32),
            pltpu.VMEM((SEG + 16,), jnp.int32),
            pltpu.VMEM((128, 128), jnp.float32),
            pltpu.VMEM((NPT, 128), jnp.float32),
            pltpu.SemaphoreType.DMA,
        ],
    )
    def k(feat_h, src_h, dst_h, out_h,
          dseg_v, sseg_v, cd_v, cs_v, rows_v, acc_v, sem):
        c = lax.axis_index("c")
        s = lax.axis_index("s")
        lo = s * NPT
        neg = jnp.full((16,), -jnp.inf, dtype=jnp.float32)
        z16i = jnp.zeros((16,), jnp.int32)
        i16 = lax.iota(jnp.int32, 16)

        def initr(i, carry):
            for j in range(8):
                acc_v[i, pl.ds(j * 16, 16)] = neg
            return carry
        lax.fori_loop(0, NPT, initr, 0)

        def initc(i, carry):
            cd_v[pl.ds(i * 16, 16)] = z16i
            cs_v[pl.ds(i * 16, 16)] = i16 + i * 16
            return carry
        lax.fori_loop(0, (SEG + 16) // 16, initc, 0)

        def seg_body(g0, carry):
            g = c * nsegh + g0
            pltpu.sync_copy(dst_h.at[pl.ds(g * SEG, SEG)], dseg_v)
            pltpu.sync_copy(src_h.at[pl.ds(g * SEG, SEG)], sseg_v)

            def comp(t, n):
                dv = dseg_v[pl.ds(t * 16, 16)]
                sv = sseg_v[pl.ds(t * 16, 16)]
                m = jnp.logical_and(dv >= lo, dv < lo + NPT)
                cum = plsc.cumsum(m.astype(jnp.int32))
                pos = cum + (n - 1)
                plsc.store_scatter(cd_v, [pos], dv, mask=m)
                plsc.store_scatter(cs_v, [pos], sv, mask=m)
                return n + jnp.max(cum)
            n = lax.fori_loop(0, SEG // 16, comp, jnp.int32(0))

            def blk(b, carry2):
                pltpu.async_copy(feat_h.at[cs_v.at[pl.ds(b * 128, 128)]],
                                 rows_v, sem).wait()

                def edge(e2, carry3):
                    e = b * 128 + e2

                    @pl.when(e < n)
                    def _():
                        gv = cd_v[pl.ds(lax.shift_right_logical(e, 4) * 16, 16)]
                        lane = lax.bitwise_and(e, 15)
                        d = jnp.max(jnp.where(i16 == lane, gv,
                                              jnp.int32(-2147483648)))
                        r = d - lo
                        for u in range(8):
                            acc_v[r, pl.ds(u * 16, 16)] = jnp.maximum(
                                acc_v[r, pl.ds(u * 16, 16)],
                                rows_v[e2, pl.ds(u * 16, 16)])
                    return carry3
                lax.fori_loop(0, 128, edge, 0)
                return carry2
            lax.fori_loop(0, (n + 127) // 128, blk, 0)
            return carry
        lax.fori_loop(0, nsegh, seg_body, 0)

        pltpu.sync_copy(acc_v, out_h.at[c, pl.ds(lo, NPT)])

    return k(feat, src1d, dst1d)


BN = 1024


def _mm_body(x_ref, w_ref, o_ref):
    o_ref[...] = jnp.dot(x_ref[...], w_ref[...],
                         preferred_element_type=jnp.float32)


def _tc_matmul(x, w):
    n, kd = x.shape
    m = w.shape[1]
    return pl.pallas_call(
        _mm_body,
        grid=(n // BN,),
        in_specs=[pl.BlockSpec((BN, kd), lambda i: (i, 0)),
                  pl.BlockSpec((kd, m), lambda i: (0, 0))],
        out_specs=pl.BlockSpec((BN, m), lambda i: (i, 0)),
        out_shape=jax.ShapeDtypeStruct((n, m), jnp.float32),
    )(x, w)


def _combine_body(sx_ref, p_ref, dp_ref, b_ref, o_ref):
    ps = p_ref[0] + p_ref[1]
    dg = jnp.maximum(jnp.sum(dp_ref[...], axis=(0, 1)), 1.0)
    ps3 = ps.reshape(BN // 128, 128, 128) / dg[:, :, None]
    o_ref[...] = jnp.tanh(sx_ref[...] + ps3.reshape(BN, 128) + b_ref[...])


def _tc_combine(sx, p, dp, base, b):
    n = sx.shape[0]
    bb = base // BN
    return pl.pallas_call(
        _combine_body,
        grid=(n // BN,),
        in_specs=[
            pl.BlockSpec((BN, 128), lambda i: (i, 0)),
            pl.BlockSpec((2, BN, 128), lambda i: (0, i, 0)),
            pl.BlockSpec((2, 16, BN // 128, 128), lambda i, b=bb: (0, 0, b + i, 0)),
            pl.BlockSpec((1, 128), lambda i: (0, 0)),
        ],
        out_specs=pl.BlockSpec((BN, 128), lambda i: (i, 0)),
        out_shape=jax.ShapeDtypeStruct((n, 128), jnp.float32),
    )(sx, p, dp, b.reshape(1, 128))


def _meanmm_body(p_ref, dp_ref, w_ref, o1_ref, o2_ref):
    ps = p_ref[0] + p_ref[1]
    dg = jnp.maximum(jnp.sum(dp_ref[...], axis=(0, 1)), 1.0)
    mfeat = (ps.reshape(BN // 128, 128, 128) / dg[:, :, None]).reshape(BN, 128)
    o1_ref[...] = mfeat
    o2_ref[...] = jnp.dot(mfeat, w_ref[...], preferred_element_type=jnp.float32)


def _tc_meanmm(p, dp, base, w, n):
    m = w.shape[1]
    bb = base // BN
    return pl.pallas_call(
        _meanmm_body,
        grid=(n // BN,),
        in_specs=[
            pl.BlockSpec((2, BN, 128), lambda i: (0, i, 0)),
            pl.BlockSpec((2, 16, BN // 128, 128), lambda i, b=bb: (0, 0, b + i, 0)),
            pl.BlockSpec((128, m), lambda i: (0, 0)),
        ],
        out_specs=[pl.BlockSpec((BN, 128), lambda i: (i, 0)),
                   pl.BlockSpec((BN, m), lambda i: (i, 0))],
        out_shape=[jax.ShapeDtypeStruct((n, 128), jnp.float32),
                   jax.ShapeDtypeStruct((n, m), jnp.float32)],
    )(p, dp, w)


def _upmm_body(a_ref, x_ref, cp_ref, wt_ref, wb_ref, o_ref):
    asum = a_ref[0] + a_ref[1]
    cnt = jnp.sum(cp_ref[...], axis=(0, 1))
    xc = (x_ref[...].reshape(BN // 128, 128, 128) * cnt[:, :, None]
          ).reshape(BN, 128)
    o_ref[...] = (jnp.dot(asum, wt_ref[...], preferred_element_type=jnp.float32)
                  + jnp.dot(xc, wb_ref[...], preferred_element_type=jnp.float32))


def _tc_upmm(p, x, cp, base, wt, wb):
    n = x.shape[0]
    m = wt.shape[1]
    bb = base // BN
    return pl.pallas_call(
        _upmm_body,
        grid=(n // BN,),
        in_specs=[
            pl.BlockSpec((2, BN, 128), lambda i: (0, i, 0)),
            pl.BlockSpec((BN, 128), lambda i: (i, 0)),
            pl.BlockSpec((2, 16, BN // 128, 128), lambda i, b=bb: (0, 0, b + i, 0)),
            pl.BlockSpec((128, m), lambda i: (0, 0)),
            pl.BlockSpec((128, m), lambda i: (0, 0)),
        ],
        out_specs=pl.BlockSpec((BN, m), lambda i: (i, 0)),
        out_shape=jax.ShapeDtypeStruct((n, m), jnp.float32),
    )(p, x, cp, wt, wb)


def _mlp_body(y_ref, w0_ref, b0_ref, w1_ref, b1_ref, o_ref):
    ym = jnp.maximum(y_ref[0], y_ref[1])
    y = jnp.where(ym > jnp.float32(-3e38), ym, 0.0)
    h = jnp.tanh(jnp.dot(y, w0_ref[...],
                         preferred_element_type=jnp.float32) + b0_ref[...])
    o_ref[...] = jnp.dot(h, w1_ref[...],
                         preferred_element_type=jnp.float32) + b1_ref[...]


def _tc_mlp(y, w0, b0, w1, b1):
    n = y.shape[1]
    return pl.pallas_call(
        _mlp_body,
        grid=(n // BN,),
        in_specs=[
            pl.BlockSpec((2, BN, 128), lambda i: (0, i, 0)),
            pl.BlockSpec((128, 128), lambda i: (0, 0)),
            pl.BlockSpec((1, 128), lambda i: (0, 0)),
            pl.BlockSpec((128, 1), lambda i: (0, 0)),
            pl.BlockSpec((1, 1), lambda i: (0, 0)),
        ],
        out_specs=pl.BlockSpec((BN, 1), lambda i: (i, 0)),
        out_shape=jax.ShapeDtypeStruct((n, 1), jnp.float32),
    )(y, w0, b0.reshape(1, 128), w1, b1.reshape(1, 1))


def _prep_edges(src, dst, nd):
    e = src.shape[0]
    ep = _ceil_to(e, EALIGN)
    pad = ep - e
    src_p = jnp.concatenate([src, jnp.zeros((pad,), jnp.int32)])
    dst_p = jnp.concatenate(
        [dst, nd + (jnp.arange(pad, dtype=jnp.int32) % 32)])
    return src_p, dst_p


def _prep_conn(src, dst):
    e = src.shape[0]
    ep = _ceil_to(e, SEG)
    pad = ep - e
    src_p = jnp.concatenate([src, jnp.zeros((pad,), jnp.int32)])
    dst_p = jnp.concatenate(
        [dst, jnp.full((pad,), 2 ** 30, dtype=jnp.int32)])
    return src_p, dst_p


def _pad_rows(a, n):
    return jnp.pad(a, ((0, n - a.shape[0]), (0, 0)))


def kernel(x, edge_lv0, d01_src, d01_dst, edge_lv1, d12_src, d12_dst,
           edge_lv2, u21_src, u21_dst, u10_src, u10_dst, conn_src, conn_dst,
           Ws0, Wn0, b0, Ws1, Wn1, b1, Ws2, Wn2, b2, Ws3, Wn3, b3,
           Ws4, Wn4, b4, Wm0, bm0, Wm1, bm1):
    NA0, NA1, NA2 = 10240, 4096, 2048
    NP1, NP2 = 3072, 1024
    NP0 = 10240

    e0 = _prep_edges(edge_lv0[0], edge_lv0[1], N0)
    e1 = _prep_edges(edge_lv1[0], edge_lv1[1], N1)
    e2 = _prep_edges(edge_lv2[0], edge_lv2[1], N2)
    ed01 = _prep_edges(d01_src, d01_dst, N1)
    ed12 = _prep_edges(d12_src, d12_dst, N2)
    eu21 = _prep_edges(u21_src, u21_dst, N1)
    eu10 = _prep_edges(u10_src, u10_dst, N0)
    cn = _prep_conn(conn_src, conn_dst)

    alldst = jnp.concatenate(
        [e0[1], ed01[1] + 10240, e1[1] + 14336, ed12[1] + 18432,
         e2[1] + 20480, eu21[1] + 22528, eu10[1] + 26624])
    DEG = _sc_degrees(alldst)
    B_LV0, B_D01, B_LV1, B_D12 = 0, 10240, 14336, 18432
    B_LV2, B_U21, B_U10 = 20480, 22528, 26624

    xp = _pad_rows(x, NP0)
    t0 = _tc_matmul(xp, jnp.concatenate([Wn0, Ws0], axis=1))
    xw0, xs0 = t0[:, :128], t0[:, 128:]
    Pa = _sc_segsum(xw0, e0[0], e0[1], NA0)
    x0 = _tc_combine(xs0, Pa, DEG, B_LV0, b0)

    Pb = _sc_segsum(x0, ed01[0], ed01[1], NA1)
    p1, pw1 = _tc_meanmm(Pb, DEG, B_D01, jnp.concatenate([Wn1, Ws1], axis=1), NP1)
    Pc = _sc_segsum(pw1[:, :128], e1[0], e1[1], NA1)
    x1 = _tc_combine(pw1[:, 128:], Pc, DEG, B_LV1, b1)

    Pd = _sc_segsum(x1, ed12[0], ed12[1], NA2)
    p2, pw2 = _tc_meanmm(Pd, DEG, B_D12, jnp.concatenate([Wn2, Ws2], axis=1), NP2)
    Pe = _sc_segsum(pw2[:, :128], e2[0], e2[1], NA2)
    x2 = _tc_combine(pw2[:, 128:], Pe, DEG, B_LV2, b2)

    Pf = _sc_segsum(x2, eu21[0], eu21[1], NA1)
    wt3 = jnp.concatenate([Wn3[:128], Ws3[:128]], axis=1)
    wb3 = jnp.concatenate([Wn3[128:], Ws3[128:]], axis=1)
    hh1 = _tc_upmm(Pf, x1, DEG, B_U21, wt3, wb3)
    Pg = _sc_segsum(hh1[:, :128], e1[0], e1[1], NA1)
    x1u = _tc_combine(hh1[:, 128:], Pg, DEG, B_LV1, b3)

    Ph = _sc_segsum(x1u, eu10[0], eu10[1], NA0)
    wt4 = jnp.concatenate([Wn4[:128], Ws4[:128]], axis=1)
    wb4 = jnp.concatenate([Wn4[128:], Ws4[128:]], axis=1)
    hh0 = _tc_upmm(Ph, x0, DEG, B_U10, wt4, wb4)
    Pi = _sc_segsum(hh0[:, :128], e0[0], e0[1], NA0)
    x0u = _tc_combine(hh0[:, 128:], Pi, DEG, B_LV0, b4)

    y = _sc_segmax(x0u, cn[0], cn[1])
    out = _tc_mlp(y, Wm0, bm0, Wm1, bm1)
    return out[:NNET]


# speedup vs baseline: 1.3025x; 1.0062x over previous
import functools

import jax
import jax.numpy as jnp
from jax import lax
from jax.experimental import pallas as pl
from jax.experimental.pallas import tpu as pltpu
from jax.experimental.pallas import tpu_sc as plsc

N0, N1, N2, NNET = 10000, 2500, 625, 8000
D = 128
NW = 32
CHUNK = 128
EALIGN = NW * CHUNK


def _ceil_to(x, m):
    return (x + m - 1) // m * m


@functools.partial(jax.jit, static_argnums=(3,))
def _sc_segsum(feat, src1d, dst1d, nd_acc):
    c_tot = src1d.shape[0] // CHUNK
    cpw = c_tot // NW
    zc = nd_acc // 16 // 64
    mesh = plsc.VectorSubcoreMesh(core_axis_name="c", subcore_axis_name="s")

    @functools.partial(
        pl.kernel,
        out_type=jax.ShapeDtypeStruct((2, nd_acc, 128), jnp.float32),
        mesh=mesh,
        compiler_params=pltpu.CompilerParams(needs_layout_passes=False),
        scratch_types=[
            pltpu.VMEM((CHUNK,), jnp.int32),
            pltpu.VMEM((CHUNK,), jnp.int32),
            pltpu.VMEM((CHUNK,), jnp.int32),
            pltpu.VMEM((CHUNK,), jnp.int32),
            pltpu.VMEM((CHUNK, 128), jnp.float32),
            pltpu.VMEM((CHUNK, 128), jnp.float32),
            pltpu.VMEM((64, 128), jnp.float32),
            pltpu.VMEM_SHARED((nd_acc, 128), jnp.float32),
            pltpu.SemaphoreType.DMA,
            pltpu.SemaphoreType.DMA,
        ],
    )
    def k(feat_h, src_h, dst_h, out_h,
          src0_v, src1_v, dst0_v, dst1_v, rows0_v, rows1_v, zero_v, acc_s,
          sem0, sem1):
        c = lax.axis_index("c")
        s = lax.axis_index("s")
        w = c * 16 + s
        z16 = jnp.zeros((16,), jnp.float32)
        ebase = w * (cpw * CHUNK)

        def zrow(i, carry):
            for j in range(8):
                zero_v[i, pl.ds(j * 16, 16)] = z16
            return carry
        lax.fori_loop(0, 64, zrow, 0)

        zb = s * (nd_acc // 16)
        for q in range(zc):
            pltpu.sync_copy(zero_v, acc_s.at[pl.ds(zb + q * 64, 64)])
        plsc.subcore_barrier()

        bufs = (rows0_v, rows1_v)
        srcs = (src0_v, src1_v)
        dsts = (dst0_v, dst1_v)
        sems = (sem0, sem1)
        pltpu.sync_copy(src_h.at[pl.ds(ebase, CHUNK)], src0_v)
        pend = {0: pltpu.async_copy(feat_h.at[src0_v], rows0_v, sem0)}
        for j in range(cpw):
            b = j % 2
            if j + 1 < cpw:
                nb = (j + 1) % 2
                pltpu.sync_copy(
                    dst_h.at[pl.ds(ebase + (j + 1) * CHUNK, CHUNK)] if False
                    else src_h.at[pl.ds(ebase + (j + 1) * CHUNK, CHUNK)],
                    srcs[nb])
                pend[j + 1] = pltpu.async_copy(
                    feat_h.at[srcs[nb]], bufs[nb], sems[nb])
            pltpu.sync_copy(dst_h.at[pl.ds(ebase + j * CHUNK, CHUNK)],
                            dsts[b])
            pend[j].wait()
            pltpu.sync_copy(bufs[b], acc_s.at[dsts[b]], add=True)
        plsc.subcore_barrier()

        orows = nd_acc // 16
        pltpu.sync_copy(acc_s.at[pl.ds(s * orows, orows)],
                        out_h.at[c, pl.ds(s * orows, orows)])

    return k(feat, src1d, dst1d)


NDEG = 36864
NDEGR = NDEG // 128


@jax.jit
def _sc_degrees(dst1d):
    c_tot = dst1d.shape[0] // CHUNK
    cpw = c_tot // NW
    mesh = plsc.VectorSubcoreMesh(core_axis_name="c", subcore_axis_name="s")

    @functools.partial(
        pl.kernel,
        out_type=jax.ShapeDtypeStruct((2, 16, NDEGR, 128), jnp.float32),
        mesh=mesh,
        compiler_params=pltpu.CompilerParams(needs_layout_passes=False),
        scratch_types=[
            pltpu.VMEM((CHUNK,), jnp.int32),
            pltpu.VMEM((NDEGR, 128), jnp.float32),
        ],
    )
    def k(dst_h, deg_h, dst_v, degl_v):
        c = lax.axis_index("c")
        s = lax.axis_index("s")
        w = c * 16 + s
        z16 = jnp.zeros((16,), jnp.float32)
        one16 = jnp.ones((16,), jnp.float32)

        def zrow(i, carry):
            for j in range(8):
                degl_v[i, pl.ds(j * 16, 16)] = z16
            return carry
        lax.fori_loop(0, NDEGR, zrow, 0)

        def body(j, carry):
            base = (w * cpw + j) * CHUNK
            pltpu.sync_copy(dst_h.at[pl.ds(base, CHUNK)], dst_v)
            for l in range(CHUNK // 16):
                dv = dst_v[pl.ds(l * 16, 16)]
                plsc.addupdate_scatter(
                    degl_v,
                    [lax.shift_right_logical(dv, 7), lax.bitwise_and(dv, 127)],
                    one16)
            return carry
        lax.fori_loop(0, cpw, body, 0)
        pltpu.sync_copy(degl_v, deg_h.at[c, s])

    return k(dst1d)


SEG = 1024
NNET_PAD = 8192
NPT = NNET_PAD // 16


@jax.jit
def _sc_segmax(feat, src1d, dst1d):
    nseg = src1d.shape[0] // SEG
    nsegh = nseg // 2
    mesh = plsc.VectorSubcoreMesh(core_axis_name="c", subcore_axis_name="s")

    @functools.partial(
        pl.kernel,
        out_type=jax.ShapeDtypeStruct((2, NNET_PAD, 128), jnp.float32),
        mesh=mesh,
        compiler_params=pltpu.CompilerParams(needs_layout_passes=False),
        scratch_types=[
            pltpu.VMEM((SEG,), jnp.int32),
            pltpu.VMEM((SEG,), jnp.int32),
            pltpu.VMEM((SEG + 16,), jnp.int32),
            pltpu.VMEM((SEG + 16,), jnp.int32),
            pltpu.VMEM((128, 128), jnp.float32),
            pltpu.VMEM((NPT, 128), jnp.float32),
            pltpu.SemaphoreType.DMA,
        ],
    )
    def k(feat_h, src_h, dst_h, out_h,
          dseg_v, sseg_v, cd_v, cs_v, rows_v, acc_v, sem):
        c = lax.axis_index("c")
        s = lax.axis_index("s")
        lo = s * NPT
        neg = jnp.full((16,), -jnp.inf, dtype=jnp.float32)
        z16i = jnp.zeros((16,), jnp.int32)
        i16 = lax.iota(jnp.int32, 16)

        def initr(i, carry):
            for j in range(8):
                acc_v[i, pl.ds(j * 16, 16)] = neg
            return carry
        lax.fori_loop(0, NPT, initr, 0)

        def initc(i, carry):
            cd_v[pl.ds(i * 16, 16)] = z16i
            cs_v[pl.ds(i * 16, 16)] = i16 + i * 16
            return carry
        lax.fori_loop(0, (SEG + 16) // 16, initc, 0)

        def seg_body(g0, carry):
            g = c * nsegh + g0
            pltpu.sync_copy(dst_h.at[pl.ds(g * SEG, SEG)], dseg_v)
            pltpu.sync_copy(src_h.at[pl.ds(g * SEG, SEG)], sseg_v)

            def comp(t, n):
                dv = dseg_v[pl.ds(t * 16, 16)]
                sv = sseg_v[pl.ds(t * 16, 16)]
                m = jnp.logical_and(dv >= lo, dv < lo + NPT)
                cum = plsc.cumsum(m.astype(jnp.int32))
                pos = cum + (n - 1)
                plsc.store_scatter(cd_v, [pos], dv, mask=m)
                plsc.store_scatter(cs_v, [pos], sv, mask=m)
                return n + jnp.max(cum)
            n = lax.fori_loop(0, SEG // 16, comp, jnp.int32(0))

            def blk(b, carry2):
                pltpu.async_copy(feat_h.at[cs_v.at[pl.ds(b * 128, 128)]],
                                 rows_v, sem).wait()

                def edge(e2, carry3):
                    e = b * 128 + e2

                    @pl.when(e < n)
                    def _():
                        gv = cd_v[pl.ds(lax.shift_right_logical(e, 4) * 16, 16)]
                        lane = lax.bitwise_and(e, 15)
                        d = jnp.max(jnp.where(i16 == lane, gv,
                                              jnp.int32(-2147483648)))
                        r = d - lo
                        for u in range(8):
                            acc_v[r, pl.ds(u * 16, 16)] = jnp.maximum(
                                acc_v[r, pl.ds(u * 16, 16)],
                                rows_v[e2, pl.ds(u * 16, 16)])
                    return carry3
                lax.fori_loop(0, 128, edge, 0)
                return carry2
            lax.fori_loop(0, (n + 127) // 128, blk, 0)
            return carry
        lax.fori_loop(0, nsegh, seg_body, 0)

        pltpu.sync_copy(acc_v, out_h.at[c, pl.ds(lo, NPT)])

    return k(feat, src1d, dst1d)


BN = 1024


def _mm_body(x_ref, w_ref, o_ref):
    o_ref[...] = jnp.dot(x_ref[...], w_ref[...],
                         preferred_element_type=jnp.float32)


def _tc_matmul(x, w):
    n, kd = x.shape
    m = w.shape[1]
    return pl.pallas_call(
        _mm_body,
        grid=(n // BN,),
        in_specs=[pl.BlockSpec((BN, kd), lambda i: (i, 0)),
                  pl.BlockSpec((kd, m), lambda i: (0, 0))],
        out_specs=pl.BlockSpec((BN, m), lambda i: (i, 0)),
        out_shape=jax.ShapeDtypeStruct((n, m), jnp.float32),
    )(x, w)


def _combine_body(sx_ref, p_ref, dp_ref, b_ref, o_ref):
    ps = p_ref[0] + p_ref[1]
    dg = jnp.maximum(jnp.sum(dp_ref[...], axis=(0, 1)), 1.0)
    ps3 = ps.reshape(BN // 128, 128, 128) / dg[:, :, None]
    o_ref[...] = jnp.tanh(sx_ref[...] + ps3.reshape(BN, 128) + b_ref[...])


def _tc_combine(sx, p, dp, base, b):
    n = sx.shape[0]
    bb = base // BN
    return pl.pallas_call(
        _combine_body,
        grid=(n // BN,),
        in_specs=[
            pl.BlockSpec((BN, 128), lambda i: (i, 0)),
            pl.BlockSpec((2, BN, 128), lambda i: (0, i, 0)),
            pl.BlockSpec((2, 16, BN // 128, 128), lambda i, b=bb: (0, 0, b + i, 0)),
            pl.BlockSpec((1, 128), lambda i: (0, 0)),
        ],
        out_specs=pl.BlockSpec((BN, 128), lambda i: (i, 0)),
        out_shape=jax.ShapeDtypeStruct((n, 128), jnp.float32),
    )(sx, p, dp, b.reshape(1, 128))


def _meanmm_body(p_ref, dp_ref, w_ref, o1_ref, o2_ref):
    ps = p_ref[0] + p_ref[1]
    dg = jnp.maximum(jnp.sum(dp_ref[...], axis=(0, 1)), 1.0)
    mfeat = (ps.reshape(BN // 128, 128, 128) / dg[:, :, None]).reshape(BN, 128)
    o1_ref[...] = mfeat
    o2_ref[...] = jnp.dot(mfeat, w_ref[...], preferred_element_type=jnp.float32)


def _tc_meanmm(p, dp, base, w, n):
    m = w.shape[1]
    bb = base // BN
    return pl.pallas_call(
        _meanmm_body,
        grid=(n // BN,),
        in_specs=[
            pl.BlockSpec((2, BN, 128), lambda i: (0, i, 0)),
            pl.BlockSpec((2, 16, BN // 128, 128), lambda i, b=bb: (0, 0, b + i, 0)),
            pl.BlockSpec((128, m), lambda i: (0, 0)),
        ],
        out_specs=[pl.BlockSpec((BN, 128), lambda i: (i, 0)),
                   pl.BlockSpec((BN, m), lambda i: (i, 0))],
        out_shape=[jax.ShapeDtypeStruct((n, 128), jnp.float32),
                   jax.ShapeDtypeStruct((n, m), jnp.float32)],
    )(p, dp, w)


def _upmm_body(a_ref, x_ref, cp_ref, wt_ref, wb_ref, o_ref):
    asum = a_ref[0] + a_ref[1]
    cnt = jnp.sum(cp_ref[...], axis=(0, 1))
    xc = (x_ref[...].reshape(BN // 128, 128, 128) * cnt[:, :, None]
          ).reshape(BN, 128)
    o_ref[...] = (jnp.dot(asum, wt_ref[...], preferred_element_type=jnp.float32)
                  + jnp.dot(xc, wb_ref[...], preferred_element_type=jnp.float32))


def _tc_upmm(p, x, cp, base, wt, wb):
    n = x.shape[0]
    m = wt.shape[1]
    bb = base // BN
    return pl.pallas_call(
        _upmm_body,
        grid=(n // BN,),
        in_specs=[
            pl.BlockSpec((2, BN, 128), lambda i: (0, i, 0)),
            pl.BlockSpec((BN, 128), lambda i: (i, 0)),
            pl.BlockSpec((2, 16, BN // 128, 128), lambda i, b=bb: (0, 0, b + i, 0)),
            pl.BlockSpec((128, m), lambda i: (0, 0)),
            pl.BlockSpec((128, m), lambda i: (0, 0)),
        ],
        out_specs=pl.BlockSpec((BN, m), lambda i: (i, 0)),
        out_shape=jax.ShapeDtypeStruct((n, m), jnp.float32),
    )(p, x, cp, wt, wb)


def _mlp_body(y_ref, w0_ref, b0_ref, w1_ref, b1_ref, o_ref):
    ym = jnp.maximum(y_ref[0], y_ref[1])
    y = jnp.where(ym > jnp.float32(-3e38), ym, 0.0)
    h = jnp.tanh(jnp.dot(y, w0_ref[...],
                         preferred_element_type=jnp.float32) + b0_ref[...])
    o_ref[...] = jnp.dot(h, w1_ref[...],
                         preferred_element_type=jnp.float32) + b1_ref[...]


def _tc_mlp(y, w0, b0, w1, b1):
    n = y.shape[1]
    return pl.pallas_call(
        _mlp_body,
        grid=(n // BN,),
        in_specs=[
            pl.BlockSpec((2, BN, 128), lambda i: (0, i, 0)),
            pl.BlockSpec((128, 128), lambda i: (0, 0)),
            pl.BlockSpec((1, 128), lambda i: (0, 0)),
            pl.BlockSpec((128, 1), lambda i: (0, 0)),
            pl.BlockSpec((1, 1), lambda i: (0, 0)),
        ],
        out_specs=pl.BlockSpec((BN, 1), lambda i: (i, 0)),
        out_shape=jax.ShapeDtypeStruct((n, 1), jnp.float32),
    )(y, w0, b0.reshape(1, 128), w1, b1.reshape(1, 1))


def _prep_edges(src, dst, nd):
    e = src.shape[0]
    ep = _ceil_to(e, EALIGN)
    pad = ep - e
    src_p = jnp.concatenate([src, jnp.zeros((pad,), jnp.int32)])
    dst_p = jnp.concatenate(
        [dst, nd + (jnp.arange(pad, dtype=jnp.int32) % 32)])
    return src_p, dst_p


def _prep_conn(src, dst):
    e = src.shape[0]
    ep = _ceil_to(e, SEG)
    pad = ep - e
    src_p = jnp.concatenate([src, jnp.zeros((pad,), jnp.int32)])
    dst_p = jnp.concatenate(
        [dst, jnp.full((pad,), 2 ** 30, dtype=jnp.int32)])
    return src_p, dst_p


def _pad_rows(a, n):
    return jnp.pad(a, ((0, n - a.shape[0]), (0, 0)))


def kernel(x, edge_lv0, d01_src, d01_dst, edge_lv1, d12_src, d12_dst,
           edge_lv2, u21_src, u21_dst, u10_src, u10_dst, conn_src, conn_dst,
           Ws0, Wn0, b0, Ws1, Wn1, b1, Ws2, Wn2, b2, Ws3, Wn3, b3,
           Ws4, Wn4, b4, Wm0, bm0, Wm1, bm1):
    NA0, NA1, NA2 = 10240, 3072, 1024
    NP1, NP2 = 3072, 1024
    NP0 = 10240

    e0 = _prep_edges(edge_lv0[0], edge_lv0[1], N0)
    e1 = _prep_edges(edge_lv1[0], edge_lv1[1], N1)
    e2 = _prep_edges(edge_lv2[0], edge_lv2[1], N2)
    ed01 = _prep_edges(d01_src, d01_dst, N1)
    ed12 = _prep_edges(d12_src, d12_dst, N2)
    eu21 = _prep_edges(u21_src, u21_dst, N1)
    eu10 = _prep_edges(u10_src, u10_dst, N0)
    cn = _prep_conn(conn_src, conn_dst)

    alldst = jnp.concatenate(
        [e0[1], ed01[1] + 10240, e1[1] + 14336, ed12[1] + 18432,
         e2[1] + 20480, eu21[1] + 22528, eu10[1] + 26624])
    DEG = _sc_degrees(alldst)
    B_LV0, B_D01, B_LV1, B_D12 = 0, 10240, 14336, 18432
    B_LV2, B_U21, B_U10 = 20480, 22528, 26624

    xp = _pad_rows(x, NP0)
    t0 = _tc_matmul(xp, jnp.concatenate([Wn0, Ws0], axis=1))
    xw0, xs0 = t0[:, :128], t0[:, 128:]
    Pa = _sc_segsum(xw0, e0[0], e0[1], NA0)
    x0 = _tc_combine(xs0, Pa, DEG, B_LV0, b0)

    Pb = _sc_segsum(x0, ed01[0], ed01[1], NA1)
    p1, pw1 = _tc_meanmm(Pb, DEG, B_D01, jnp.concatenate([Wn1, Ws1], axis=1), NP1)
    Pc = _sc_segsum(pw1[:, :128], e1[0], e1[1], NA1)
    x1 = _tc_combine(pw1[:, 128:], Pc, DEG, B_LV1, b1)

    Pd = _sc_segsum(x1, ed12[0], ed12[1], NA2)
    p2, pw2 = _tc_meanmm(Pd, DEG, B_D12, jnp.concatenate([Wn2, Ws2], axis=1), NP2)
    Pe = _sc_segsum(pw2[:, :128], e2[0], e2[1], NA2)
    x2 = _tc_combine(pw2[:, 128:], Pe, DEG, B_LV2, b2)

    Pf = _sc_segsum(x2, eu21[0], eu21[1], NA1)
    wt3 = jnp.concatenate([Wn3[:128], Ws3[:128]], axis=1)
    wb3 = jnp.concatenate([Wn3[128:], Ws3[128:]], axis=1)
    hh1 = _tc_upmm(Pf, x1, DEG, B_U21, wt3, wb3)
    Pg = _sc_segsum(hh1[:, :128], e1[0], e1[1], NA1)
    x1u = _tc_combine(hh1[:, 128:], Pg, DEG, B_LV1, b3)

    Ph = _sc_segsum(x1u, eu10[0], eu10[1], NA0)
    wt4 = jnp.concatenate([Wn4[:128], Ws4[:128]], axis=1)
    wb4 = jnp.concatenate([Wn4[128:], Ws4[128:]], axis=1)
    hh0 = _tc_upmm(Ph, x0, DEG, B_U10, wt4, wb4)
    Pi = _sc_segsum(hh0[:, :128], e0[0], e0[1], NA0)
    x0u = _tc_combine(hh0[:, 128:], Pi, DEG, B_LV0, b4)

    y = _sc_segmax(x0u, cn[0], cn[1])
    out = _tc_mlp(y, Wm0, bm0, Wm1, bm1)
    return out[:NNET]
